# baseline (speedup 1.0000x reference)
import functools

import jax
import jax.numpy as jnp
from jax.experimental import pallas as pl
from jax.experimental.pallas import tpu as pltpu

Q_TOTAL = 1024
N_KEYS = 100000
DIM = 64
K_KEEP = 33

QB = 16
CPAD = 100352
CW = 1024
NCH = CPAD // CW
PREP_B = 2048
NEG = float("-inf")
BIGCOL = 2**30


def _prep_body(xn_ref, out_ref):
    v = xn_ref[...]
    nrm = jnp.sqrt(jnp.sum(v * v, axis=1, keepdims=True))
    out_ref[...] = (v / jnp.maximum(nrm, 1e-12)).T


def _main_body(q_ref, xnn_ref, out_ref, s_ref, orig_ref):
    q = q_ref[...]
    qn = q / jnp.maximum(jnp.sqrt(jnp.sum(q * q, axis=1, keepdims=True)), 1e-12)

    def col_iota(c):
        return c * CW + jax.lax.broadcasted_iota(jnp.int32, (QB, CW), 1)

    def mm(c, carry):
        blk = xnn_ref[:, pl.ds(c * CW, CW)]
        sim = jax.lax.dot_general(
            qn, blk, (((1,), (0,)), ((), ())),
            preferred_element_type=jnp.float32)
        sim = jnp.where(col_iota(c) < N_KEYS, sim, NEG)
        s_ref[:, pl.ds(c * CW, CW)] = sim
        orig_ref[:, pl.ds(c * CW, CW)] = sim
        return carry

    jax.lax.fori_loop(0, NCH, mm, 0, unroll=False)

    def extract(j, carry):
        def p1(c, m):
            chunk = s_ref[:, pl.ds(c * CW, CW)]
            return jnp.maximum(m, jnp.max(chunk, axis=1, keepdims=True))

        m = jax.lax.fori_loop(0, NCH, p1,
                              jnp.full((QB, 1), NEG, jnp.float32),
                              unroll=False)

        def p2(c, amc):
            chunk = s_ref[:, pl.ds(c * CW, CW)]
            cand = jnp.where(chunk == m, col_iota(c), BIGCOL)
            return jnp.minimum(amc, jnp.min(cand, axis=1, keepdims=True))

        amc = jax.lax.fori_loop(0, NCH, p2,
                                jnp.full((QB, 1), BIGCOL, jnp.int32),
                                unroll=False)

        def p3(c, carry2):
            chunk = s_ref[:, pl.ds(c * CW, CW)]
            s_ref[:, pl.ds(c * CW, CW)] = jnp.where(
                col_iota(c) == amc, NEG, chunk)
            return carry2

        jax.lax.fori_loop(0, NCH, p3, 0, unroll=False)
        return carry

    jax.lax.fori_loop(0, K_KEEP, extract, 0, unroll=False)

    orig = orig_ref[:, :N_KEYS]
    out_ref[...] = jnp.where(s_ref[:, :N_KEYS] == orig, NEG, orig)


@functools.partial(jax.jit, static_argnames=("interpret",))
def kernel(x, x_n, interpret=False):
    xp = jnp.pad(x_n, ((0, CPAD - N_KEYS), (0, 0)))
    xnn = pl.pallas_call(
        _prep_body,
        grid=(CPAD // PREP_B,),
        in_specs=[pl.BlockSpec((PREP_B, DIM), lambda i: (i, 0))],
        out_specs=pl.BlockSpec((DIM, PREP_B), lambda i: (0, i)),
        out_shape=jax.ShapeDtypeStruct((DIM, CPAD), jnp.float32),
        interpret=interpret,
    )(xp)

    out = pl.pallas_call(
        _main_body,
        grid=(Q_TOTAL // QB,),
        in_specs=[
            pl.BlockSpec((QB, DIM), lambda i: (i, 0)),
            pl.BlockSpec((DIM, CPAD), lambda i: (0, 0)),
        ],
        out_specs=pl.BlockSpec((QB, N_KEYS), lambda i: (i, 0)),
        out_shape=jax.ShapeDtypeStruct((Q_TOTAL, N_KEYS), jnp.float32),
        scratch_shapes=[
            pltpu.VMEM((QB, CPAD), jnp.float32),
            pltpu.VMEM((QB, CPAD), jnp.float32),
        ],
        interpret=interpret,
    )(x, xnn)
    return out


# speedup vs baseline: 4.7663x; 4.7663x over previous
import functools

import jax
import jax.numpy as jnp
from jax import lax
from jax.experimental import pallas as pl
from jax.experimental.pallas import tpu as pltpu
from jax.experimental.pallas import tpu_sc as plsc

Q_TOTAL = 1024
N_KEYS = 100000
DIM = 64
K_KEEP = 33
K_PAD = 36

QB = 64
CPAD = 100352
CW = 1024
NCH = CPAD // CW
NCHPAD = 128
PREP_B = 2048
NEG = float("-inf")
BIGCOL = 2**30

NW = 32
RPT = Q_TOTAL // NW
SHARD = RPT * N_KEYS
FILL_CHUNK = 64000
N_IDX_ROWS = (RPT * K_PAD) // 128


def _prep_body(xn_ref, out_ref):
    v = xn_ref[...]
    nrm = jnp.sqrt(jnp.sum(v * v, axis=1, keepdims=True))
    out_ref[...] = (v / jnp.maximum(nrm, 1e-12)).T


def _topk_body(q_ref, xnn_ref, vals_ref, cols_ref, s_ref, cm_ref, amc_ref):
    q = q_ref[...]
    qn = q / jnp.maximum(jnp.sqrt(jnp.sum(q * q, axis=1, keepdims=True)), 1e-12)

    cm_ref[...] = jnp.full((QB, NCHPAD), NEG, jnp.float32)
    vals_ref[...] = jnp.full((QB, K_PAD), NEG, jnp.float32)
    cols_ref[...] = jnp.zeros((QB, K_PAD), jnp.int32)

    lcol = jax.lax.broadcasted_iota(jnp.int32, (QB, CW), 1)
    li = jax.lax.broadcasted_iota(jnp.int32, (QB, NCHPAD), 1)
    kiota = jax.lax.broadcasted_iota(jnp.int32, (QB, K_PAD), 1)

    def mm(c, carry):
        off = pl.multiple_of(c * CW, CW)
        blk = xnn_ref[:, pl.ds(off, CW)]
        sim = jax.lax.dot_general(
            qn, blk, (((1,), (0,)), ((), ())),
            preferred_element_type=jnp.float32)
        sim = jnp.where(c * CW + lcol < N_KEYS, sim, NEG)
        s_ref[:, pl.ds(off, CW)] = sim
        mx = jnp.max(sim, axis=1, keepdims=True)
        cm_ref[...] = jnp.where(li == c, mx, cm_ref[...])
        return carry

    lax.fori_loop(0, NCH, mm, 0, unroll=False)

    def extract(j, carry):
        cmv = cm_ref[...]
        m = jnp.max(cmv, axis=1, keepdims=True)
        csel = jnp.min(jnp.where(cmv == m, li, BIGCOL), axis=1, keepdims=True)
        amc_ref[...] = jnp.full((QB, 1), BIGCOL, jnp.int32)

        def chunk_step(c, carry2):
            @pl.when(jnp.any(csel == c))
            def _():
                off = pl.multiple_of(c * CW, CW)
                rows = csel == c
                chunk = s_ref[:, pl.ds(off, CW)]
                eq = jnp.logical_and(chunk == m, rows)
                am = jnp.min(jnp.where(eq, lcol, BIGCOL), axis=1,
                             keepdims=True)
                newchunk = jnp.where(lcol == am, NEG, chunk)
                s_ref[:, pl.ds(off, CW)] = newchunk
                nm = jnp.max(newchunk, axis=1, keepdims=True)
                cm_ref[...] = jnp.where(
                    jnp.logical_and(li == c, rows), nm, cm_ref[...])
                amc_ref[...] = jnp.where(rows, c * CW + am, amc_ref[...])
            return carry2

        lax.fori_loop(0, NCH, chunk_step, 0, unroll=False)
        vals_ref[...] = jnp.where(kiota == j, m, vals_ref[...])
        cols_ref[...] = jnp.where(kiota == j, amc_ref[...], cols_ref[...])
        return carry

    lax.fori_loop(0, K_KEEP, extract, 0, unroll=False)

    vals_ref[...] = jnp.where(kiota >= K_KEEP, vals_ref[:, 0:1], vals_ref[...])
    cols_ref[...] = jnp.where(kiota >= K_KEEP, cols_ref[:, 0:1], cols_ref[...])


def _topk_call(x, xnn, interpret=False):
    return pl.pallas_call(
        _topk_body,
        grid=(Q_TOTAL // QB,),
        in_specs=[
            pl.BlockSpec((QB, DIM), lambda i: (i, 0)),
            pl.BlockSpec((DIM, CPAD), lambda i: (0, 0)),
        ],
        out_specs=[
            pl.BlockSpec((QB, K_PAD), lambda i: (i, 0)),
            pl.BlockSpec((QB, K_PAD), lambda i: (i, 0)),
        ],
        out_shape=[
            jax.ShapeDtypeStruct((Q_TOTAL, K_PAD), jnp.float32),
            jax.ShapeDtypeStruct((Q_TOTAL, K_PAD), jnp.int32),
        ],
        scratch_shapes=[
            pltpu.VMEM((QB, CPAD), jnp.float32),
            pltpu.VMEM((QB, NCHPAD), jnp.float32),
            pltpu.VMEM((QB, 1), jnp.int32),
        ],
        interpret=interpret,
    )(x, xnn)


def _prep_call(x_n, interpret=False):
    xp = jnp.pad(x_n, ((0, CPAD - N_KEYS), (0, 0)))
    return pl.pallas_call(
        _prep_body,
        grid=(CPAD // PREP_B,),
        in_specs=[pl.BlockSpec((PREP_B, DIM), lambda i: (i, 0))],
        out_specs=pl.BlockSpec((DIM, PREP_B), lambda i: (0, i)),
        out_shape=jax.ShapeDtypeStruct((DIM, CPAD), jnp.float32),
        interpret=interpret,
    )(xp)


def _make_scatter_kernel():
    mesh = plsc.VectorSubcoreMesh(core_axis_name="c", subcore_axis_name="s")

    @functools.partial(
        pl.kernel,
        out_type=jax.ShapeDtypeStruct((Q_TOTAL * N_KEYS,), jnp.float32),
        mesh=mesh,
        scratch_types=[
            pltpu.VMEM((FILL_CHUNK,), jnp.float32),
            pltpu.VMEM((N_IDX_ROWS, 128), jnp.int32),
            pltpu.VMEM((N_IDX_ROWS, 128), jnp.float32),
            pltpu.SemaphoreType.DMA,
        ],
    )
    def scatter_kernel(vals_hbm, idx_hbm, out_hbm, neg_v, idx_v, val_v, sem):
        wid = lax.axis_index("s") * 2 + lax.axis_index("c")
        base = wid * SHARD

        def fill_neg(i, carry):
            neg_v[pl.ds(i * 16, 16)] = jnp.full((16,), NEG, jnp.float32)
            return carry

        lax.fori_loop(0, FILL_CHUNK // 16, fill_neg, 0, unroll=False)

        def fill_out(k, carry):
            pltpu.sync_copy(neg_v,
                            out_hbm.at[pl.ds(base + k * FILL_CHUNK,
                                             FILL_CHUNK)])
            return carry

        lax.fori_loop(0, SHARD // FILL_CHUNK, fill_out, 0, unroll=False)

        pltpu.sync_copy(idx_hbm.at[wid], idx_v)
        pltpu.sync_copy(vals_hbm.at[wid], val_v)
        for j in range(N_IDX_ROWS):
            pltpu.async_copy(val_v.at[j], out_hbm.at[idx_v.at[j]], sem).wait()

    return scatter_kernel


def kernel(x, x_n):
    xnn = _prep_call(x_n)
    vals, cols = _topk_call(x, xnn)
    rows = jax.lax.broadcasted_iota(jnp.int32, (Q_TOTAL, K_PAD), 0)
    flat = rows * N_KEYS + cols
    vals3 = vals.reshape(NW, N_IDX_ROWS, 128)
    idx3 = flat.reshape(NW, N_IDX_ROWS, 128)
    out_flat = _make_scatter_kernel()(vals3, idx3)
    return out_flat.reshape(Q_TOTAL, N_KEYS)


# speedup vs baseline: 7.1645x; 1.5032x over previous
import functools

import jax
import jax.numpy as jnp
from jax import lax
from jax.experimental import pallas as pl
from jax.experimental.pallas import tpu as pltpu
from jax.experimental.pallas import tpu_sc as plsc

Q_TOTAL = 1024
N_KEYS = 100000
DIM = 64
K_KEEP = 33
K_PAD = 36

QB = 64
CPAD = 100352
CW = 512
NCH = CPAD // CW
NCHPAD = 256
MW = 2048
NMM = CPAD // MW
PREP_B = 2048
NEG = float("-inf")
BIGCOL = 2**30

NW = 32
RPT = Q_TOTAL // NW
SHARD = RPT * N_KEYS
FILL_CHUNK = 64000
N_IDX_ROWS = (RPT * K_PAD) // 128


def _prep_body(xn_ref, out_ref):
    v = xn_ref[...]
    nrm = jnp.sqrt(jnp.sum(v * v, axis=1, keepdims=True))
    out_ref[...] = (v / jnp.maximum(nrm, 1e-12)).T


def _topk_body(q_ref, xnn_ref, vals_ref, cols_ref, s_ref, cm_ref, amc_ref,
               cselv_ref, csels_ref, sem):
    q = q_ref[...]
    qn = q / jnp.maximum(jnp.sqrt(jnp.sum(q * q, axis=1, keepdims=True)), 1e-12)

    cm_ref[...] = jnp.full((QB, NCHPAD), NEG, jnp.float32)
    vals_ref[...] = jnp.full((QB, K_PAD), NEG, jnp.float32)
    cols_ref[...] = jnp.zeros((QB, K_PAD), jnp.int32)

    mcol = jax.lax.broadcasted_iota(jnp.int32, (QB, MW), 1)
    li = jax.lax.broadcasted_iota(jnp.int32, (QB, NCHPAD), 1)
    kiota = jax.lax.broadcasted_iota(jnp.int32, (QB, K_PAD), 1)
    lcol1 = jax.lax.broadcasted_iota(jnp.int32, (1, CW), 1)
    li1 = jax.lax.broadcasted_iota(jnp.int32, (1, NCHPAD), 1)

    def mm(c, carry):
        off = pl.multiple_of(c * MW, MW)
        blk = xnn_ref[:, pl.ds(off, MW)]
        sim = jax.lax.dot_general(
            qn, blk, (((1,), (0,)), ((), ())),
            preferred_element_type=jnp.float32)
        sim = jnp.where(c * MW + mcol < N_KEYS, sim, NEG)
        s_ref[:, pl.ds(off, MW)] = sim
        cmu = cm_ref[...]
        for sub in range(MW // CW):
            mx = jnp.max(sim[:, sub * CW:(sub + 1) * CW], axis=1,
                         keepdims=True)
            cmu = jnp.where(li == c * (MW // CW) + sub, mx, cmu)
        cm_ref[...] = cmu
        return carry

    lax.fori_loop(0, NMM, mm, 0, unroll=False)

    def extract(j, carry):
        cmv = cm_ref[...]
        m = jnp.max(cmv, axis=1, keepdims=True)
        csel = jnp.min(jnp.where(cmv == m, li, BIGCOL), axis=1, keepdims=True)
        cselv_ref[...] = csel
        pltpu.make_async_copy(cselv_ref, csels_ref, sem).start()
        pltpu.make_async_copy(cselv_ref, csels_ref, sem).wait()

        for r in range(QB):
            c_r = csels_ref[r, 0]
            off = pl.multiple_of(c_r * CW, CW)
            rowchunk = s_ref[r:r + 1, pl.ds(off, CW)]
            mr = m[r:r + 1, :]
            eq = rowchunk == mr
            am = jnp.min(jnp.where(eq, lcol1, BIGCOL), axis=1, keepdims=True)
            newchunk = jnp.where(lcol1 == am, NEG, rowchunk)
            s_ref[r:r + 1, pl.ds(off, CW)] = newchunk
            nm = jnp.max(newchunk, axis=1, keepdims=True)
            cm_ref[r:r + 1, :] = jnp.where(li1 == c_r, nm, cm_ref[r:r + 1, :])
            amc_ref[r:r + 1, :] = c_r * CW + am

        vals_ref[...] = jnp.where(kiota == j, m, vals_ref[...])
        cols_ref[...] = jnp.where(kiota == j, amc_ref[...], cols_ref[...])
        return carry

    lax.fori_loop(0, K_KEEP, extract, 0, unroll=False)

    vals_ref[...] = jnp.where(kiota >= K_KEEP, vals_ref[:, 0:1], vals_ref[...])
    cols_ref[...] = jnp.where(kiota >= K_KEEP, cols_ref[:, 0:1], cols_ref[...])


def _topk_call(x, xnn, interpret=False):
    return pl.pallas_call(
        _topk_body,
        grid=(Q_TOTAL // QB,),
        in_specs=[
            pl.BlockSpec((QB, DIM), lambda i: (i, 0)),
            pl.BlockSpec((DIM, CPAD), lambda i: (0, 0)),
        ],
        out_specs=[
            pl.BlockSpec((QB, K_PAD), lambda i: (i, 0)),
            pl.BlockSpec((QB, K_PAD), lambda i: (i, 0)),
        ],
        out_shape=[
            jax.ShapeDtypeStruct((Q_TOTAL, K_PAD), jnp.float32),
            jax.ShapeDtypeStruct((Q_TOTAL, K_PAD), jnp.int32),
        ],
        scratch_shapes=[
            pltpu.VMEM((QB, CPAD), jnp.float32),
            pltpu.VMEM((QB, NCHPAD), jnp.float32),
            pltpu.VMEM((QB, 1), jnp.int32),
            pltpu.VMEM((QB, 1), jnp.int32),
            pltpu.SMEM((QB, 1), jnp.int32),
            pltpu.SemaphoreType.DMA,
        ],
        interpret=interpret,
    )(x, xnn)


def _prep_call(x_n, interpret=False):
    xp = jnp.pad(x_n, ((0, CPAD - N_KEYS), (0, 0)))
    return pl.pallas_call(
        _prep_body,
        grid=(CPAD // PREP_B,),
        in_specs=[pl.BlockSpec((PREP_B, DIM), lambda i: (i, 0))],
        out_specs=pl.BlockSpec((DIM, PREP_B), lambda i: (0, i)),
        out_shape=jax.ShapeDtypeStruct((DIM, CPAD), jnp.float32),
        interpret=interpret,
    )(xp)


def _make_scatter_kernel():
    mesh = plsc.VectorSubcoreMesh(core_axis_name="c", subcore_axis_name="s")

    @functools.partial(
        pl.kernel,
        out_type=jax.ShapeDtypeStruct((Q_TOTAL * N_KEYS,), jnp.float32),
        mesh=mesh,
        scratch_types=[
            pltpu.VMEM((FILL_CHUNK,), jnp.float32),
            pltpu.VMEM((N_IDX_ROWS, 128), jnp.int32),
            pltpu.VMEM((N_IDX_ROWS, 128), jnp.float32),
            pltpu.SemaphoreType.DMA,
        ],
    )
    def scatter_kernel(vals_hbm, idx_hbm, out_hbm, neg_v, idx_v, val_v, sem):
        wid = lax.axis_index("s") * 2 + lax.axis_index("c")
        base = wid * SHARD

        def fill_neg(i, carry):
            neg_v[pl.ds(i * 16, 16)] = jnp.full((16,), NEG, jnp.float32)
            return carry

        lax.fori_loop(0, FILL_CHUNK // 16, fill_neg, 0, unroll=False)

        def fill_out(k, carry):
            pltpu.sync_copy(neg_v,
                            out_hbm.at[pl.ds(base + k * FILL_CHUNK,
                                             FILL_CHUNK)])
            return carry

        lax.fori_loop(0, SHARD // FILL_CHUNK, fill_out, 0, unroll=False)

        pltpu.sync_copy(idx_hbm.at[wid], idx_v)
        pltpu.sync_copy(vals_hbm.at[wid], val_v)
        for j in range(N_IDX_ROWS):
            pltpu.async_copy(val_v.at[j], out_hbm.at[idx_v.at[j]], sem).wait()

    return scatter_kernel


def kernel(x, x_n):
    xnn = _prep_call(x_n)
    vals, cols = _topk_call(x, xnn)
    rows = jax.lax.broadcasted_iota(jnp.int32, (Q_TOTAL, K_PAD), 0)
    flat = rows * N_KEYS + cols
    vals3 = vals.reshape(NW, N_IDX_ROWS, 128)
    idx3 = flat.reshape(NW, N_IDX_ROWS, 128)
    out_flat = _make_scatter_kernel()(vals3, idx3)
    return out_flat.reshape(Q_TOTAL, N_KEYS)


# speedup vs baseline: 18.4630x; 2.5770x over previous
import functools

import jax
import jax.numpy as jnp
from jax import lax
from jax.experimental import pallas as pl
from jax.experimental.pallas import tpu as pltpu
from jax.experimental.pallas import tpu_sc as plsc

Q_TOTAL = 1024
N_KEYS = 100000
DIM = 64
K_KEEP = 33
K_PAD = 36

QB = 64
CPAD = 100352
CW = 512
NCH = CPAD // CW
NCHPAD = 256
MW = 2048
NMM = CPAD // MW
PREP_B = 2048
NEG = float("-inf")
BIGCOL = 2**30

NW = 32
RPT = Q_TOTAL // NW
SHARD = RPT * N_KEYS
FILL_CHUNK = 64000
N_IDX_ROWS = (RPT * K_PAD) // 128


def _prep_body(xn_ref, out_ref):
    v = xn_ref[...]
    nrm = jnp.sqrt(jnp.sum(v * v, axis=1, keepdims=True))
    out_ref[...] = (v / jnp.maximum(nrm, 1e-12)).T


def _topk_body(q_ref, xnn_ref, vals_ref, cols_ref, s_ref, cm_ref, amc_ref,
               nm_ref, cselv_ref, csels_ref, sem):
    q = q_ref[...]
    qn = q / jnp.maximum(jnp.sqrt(jnp.sum(q * q, axis=1, keepdims=True)), 1e-12)

    cm_ref[...] = jnp.full((QB, NCHPAD), NEG, jnp.float32)
    vals_ref[...] = jnp.full((QB, K_PAD), NEG, jnp.float32)
    cols_ref[...] = jnp.zeros((QB, K_PAD), jnp.int32)

    mcol = jax.lax.broadcasted_iota(jnp.int32, (QB, MW), 1)
    li = jax.lax.broadcasted_iota(jnp.int32, (QB, NCHPAD), 1)
    kiota = jax.lax.broadcasted_iota(jnp.int32, (QB, K_PAD), 1)
    lcol1 = jax.lax.broadcasted_iota(jnp.int32, (1, CW), 1)
    li1 = jax.lax.broadcasted_iota(jnp.int32, (1, NCHPAD), 1)

    def mm(c, carry):
        off = pl.multiple_of(c * MW, MW)
        blk = xnn_ref[:, pl.ds(off, MW)]
        sim = jax.lax.dot_general(
            qn, blk, (((1,), (0,)), ((), ())),
            preferred_element_type=jnp.float32)
        sim = jnp.where(c * MW + mcol < N_KEYS, sim, NEG)
        s_ref[:, pl.ds(off, MW)] = sim
        cmu = cm_ref[...]
        for sub in range(MW // CW):
            mx = jnp.max(sim[:, sub * CW:(sub + 1) * CW], axis=1,
                         keepdims=True)
            cmu = jnp.where(li == c * (MW // CW) + sub, mx, cmu)
        cm_ref[...] = cmu
        return carry

    lax.fori_loop(0, NMM, mm, 0, unroll=False)

    def extract(j, carry):
        cmv = cm_ref[...]
        m = jnp.max(cmv, axis=1, keepdims=True)
        csel = jnp.min(jnp.where(cmv == m, li, BIGCOL), axis=1, keepdims=True)
        cselv_ref[...] = csel
        pltpu.make_async_copy(cselv_ref, csels_ref, sem).start()
        pltpu.make_async_copy(cselv_ref, csels_ref, sem).wait()

        G = 8
        for gi in range(QB // G):
            offs, chunks = [], []
            for k in range(G):
                r = gi * G + k
                c_r = csels_ref[r, 0]
                off = pl.multiple_of(c_r * CW, CW)
                offs.append(off)
                chunks.append(s_ref[r:r + 1, pl.ds(off, CW)])
            news, nms, amcs = [], [], []
            for k in range(G):
                r = gi * G + k
                rowchunk = chunks[k]
                eq = rowchunk == m[r:r + 1, :]
                am = jnp.min(jnp.where(eq, lcol1, BIGCOL), axis=1,
                             keepdims=True)
                newchunk = jnp.where(lcol1 == am, NEG, rowchunk)
                news.append(newchunk)
                nms.append(jnp.max(newchunk, axis=1, keepdims=True))
                amcs.append(am)
            for k in range(G):
                r = gi * G + k
                s_ref[r:r + 1, pl.ds(offs[k], CW)] = news[k]
                nm_ref[r:r + 1, :] = nms[k]
                amc_ref[r:r + 1, :] = offs[k] + amcs[k]

        cm_ref[...] = jnp.where(li == csel, nm_ref[...], cm_ref[...])
        vals_ref[...] = jnp.where(kiota == j, m, vals_ref[...])
        cols_ref[...] = jnp.where(kiota == j, amc_ref[...], cols_ref[...])
        return carry

    lax.fori_loop(0, K_KEEP, extract, 0, unroll=False)

    vals_ref[...] = jnp.where(kiota >= K_KEEP, vals_ref[:, 0:1], vals_ref[...])
    cols_ref[...] = jnp.where(kiota >= K_KEEP, cols_ref[:, 0:1], cols_ref[...])


def _topk_call(x, xnn, interpret=False):
    return pl.pallas_call(
        _topk_body,
        grid=(Q_TOTAL // QB,),
        in_specs=[
            pl.BlockSpec((QB, DIM), lambda i: (i, 0)),
            pl.BlockSpec((DIM, CPAD), lambda i: (0, 0)),
        ],
        out_specs=[
            pl.BlockSpec((QB, K_PAD), lambda i: (i, 0)),
            pl.BlockSpec((QB, K_PAD), lambda i: (i, 0)),
        ],
        out_shape=[
            jax.ShapeDtypeStruct((Q_TOTAL, K_PAD), jnp.float32),
            jax.ShapeDtypeStruct((Q_TOTAL, K_PAD), jnp.int32),
        ],
        scratch_shapes=[
            pltpu.VMEM((QB, CPAD), jnp.float32),
            pltpu.VMEM((QB, NCHPAD), jnp.float32),
            pltpu.VMEM((QB, 1), jnp.int32),
            pltpu.VMEM((QB, 1), jnp.float32),
            pltpu.VMEM((QB, 1), jnp.int32),
            pltpu.SMEM((QB, 1), jnp.int32),
            pltpu.SemaphoreType.DMA,
        ],
        interpret=interpret,
    )(x, xnn)


def _prep_call(x_n, interpret=False):
    xp = jnp.pad(x_n, ((0, CPAD - N_KEYS), (0, 0)))
    return pl.pallas_call(
        _prep_body,
        grid=(CPAD // PREP_B,),
        in_specs=[pl.BlockSpec((PREP_B, DIM), lambda i: (i, 0))],
        out_specs=pl.BlockSpec((DIM, PREP_B), lambda i: (0, i)),
        out_shape=jax.ShapeDtypeStruct((DIM, CPAD), jnp.float32),
        interpret=interpret,
    )(xp)


def _make_scatter_kernel():
    mesh = plsc.VectorSubcoreMesh(core_axis_name="c", subcore_axis_name="s")

    @functools.partial(
        pl.kernel,
        out_type=jax.ShapeDtypeStruct((Q_TOTAL * N_KEYS,), jnp.float32),
        mesh=mesh,
        scratch_types=[
            pltpu.VMEM((FILL_CHUNK,), jnp.float32),
            pltpu.VMEM((N_IDX_ROWS, 128), jnp.int32),
            pltpu.VMEM((N_IDX_ROWS, 128), jnp.float32),
            pltpu.SemaphoreType.DMA,
        ],
    )
    def scatter_kernel(vals_hbm, idx_hbm, out_hbm, neg_v, idx_v, val_v, sem):
        wid = lax.axis_index("s") * 2 + lax.axis_index("c")
        base = wid * SHARD

        def fill_neg(i, carry):
            neg_v[pl.ds(i * 16, 16)] = jnp.full((16,), NEG, jnp.float32)
            return carry

        lax.fori_loop(0, FILL_CHUNK // 16, fill_neg, 0, unroll=False)

        def fill_out(k, carry):
            pltpu.sync_copy(neg_v,
                            out_hbm.at[pl.ds(base + k * FILL_CHUNK,
                                             FILL_CHUNK)])
            return carry

        lax.fori_loop(0, SHARD // FILL_CHUNK, fill_out, 0, unroll=False)

        pltpu.sync_copy(idx_hbm.at[wid], idx_v)
        pltpu.sync_copy(vals_hbm.at[wid], val_v)
        for j in range(N_IDX_ROWS):
            pltpu.async_copy(val_v.at[j], out_hbm.at[idx_v.at[j]], sem).wait()

    return scatter_kernel


def kernel(x, x_n):
    xnn = _prep_call(x_n)
    vals, cols = _topk_call(x, xnn)
    rows = jax.lax.broadcasted_iota(jnp.int32, (Q_TOTAL, K_PAD), 0)
    flat = rows * N_KEYS + cols
    vals3 = vals.reshape(NW, N_IDX_ROWS, 128)
    idx3 = flat.reshape(NW, N_IDX_ROWS, 128)
    out_flat = _make_scatter_kernel()(vals3, idx3)
    return out_flat.reshape(Q_TOTAL, N_KEYS)


# speedup vs baseline: 23.9523x; 1.2973x over previous
import functools

import jax
import jax.numpy as jnp
from jax import lax
from jax.experimental import pallas as pl
from jax.experimental.pallas import tpu as pltpu
from jax.experimental.pallas import tpu_sc as plsc

Q_TOTAL = 1024
N_KEYS = 100000
DIM = 64
K_KEEP = 33
K_PAD = 48

QB = 64
CPAD = 100352
CW = 1024
NCH = CPAD // CW
NCHPAD = 128
MW = 2048
NMM = CPAD // MW
FOLD = 8
SW = CPAD // FOLD
S3R = QB * FOLD
PREP_B = 2048
NEG = float("-inf")
BIGCOL = 2**30

NW = 32
RPT = Q_TOTAL // NW
PERT = RPT * K_PAD


def _prep_body(xn_ref, out_ref):
    v = xn_ref[...]
    nrm = jnp.sqrt(jnp.sum(v * v, axis=1, keepdims=True))
    out_ref[...] = (v / jnp.maximum(nrm, 1e-12)).T


def _topk_body(q_ref, xnn_ref, vals_ref, cols_ref, s_ref, cm_ref, amc_ref,
               nm_ref, cselv_ref, csels_ref, sem):
    q = q_ref[...]
    qn = q / jnp.maximum(jnp.sqrt(jnp.sum(q * q, axis=1, keepdims=True)), 1e-12)

    cm_ref[...] = jnp.full((QB, NCHPAD), NEG, jnp.float32)
    vals_ref[...] = jnp.full((QB, K_PAD), NEG, jnp.float32)
    cols_ref[...] = jnp.zeros((QB, K_PAD), jnp.int32)

    mcol = jax.lax.broadcasted_iota(jnp.int32, (QB, MW), 1)
    li = jax.lax.broadcasted_iota(jnp.int32, (QB, NCHPAD), 1)
    kiota = jax.lax.broadcasted_iota(jnp.int32, (QB, K_PAD), 1)
    iota8 = (jax.lax.broadcasted_iota(jnp.int32, (FOLD, 128), 0) * 128
             + jax.lax.broadcasted_iota(jnp.int32, (FOLD, 128), 1))
    FCH = MW // (FOLD * 128)

    def mm(c, carry):
        off = pl.multiple_of(c * MW, MW)
        blk = xnn_ref[:, pl.ds(off, MW)]
        sim = jax.lax.dot_general(
            qn, blk, (((1,), (0,)), ((), ())),
            preferred_element_type=jnp.float32)
        sim = jnp.where(c * MW + mcol < N_KEYS, sim, NEG)
        folded = sim.reshape(QB, FCH, FOLD, 128).transpose(0, 2, 1, 3)
        s_ref[:, pl.ds(pl.multiple_of(c * (MW // FOLD), MW // FOLD),
                       MW // FOLD)] = folded.reshape(S3R, MW // FOLD)
        cmu = cm_ref[...]
        for sub in range(MW // CW):
            mx = jnp.max(sim[:, sub * CW:(sub + 1) * CW], axis=1,
                         keepdims=True)
            cmu = jnp.where(li == c * (MW // CW) + sub, mx, cmu)
        cm_ref[...] = cmu
        return carry

    lax.fori_loop(0, NMM, mm, 0, unroll=False)

    def extract(j, carry):
        cmv = cm_ref[...]
        m = jnp.max(cmv, axis=1, keepdims=True)
        csel = jnp.min(jnp.where(cmv == m, li, BIGCOL), axis=1, keepdims=True)
        cselv_ref[...] = csel
        pltpu.make_async_copy(cselv_ref, csels_ref, sem).start()
        pltpu.make_async_copy(cselv_ref, csels_ref, sem).wait()

        G = 16
        for gi in range(QB // G):
            offs, chunks = [], []
            for k in range(G):
                r = gi * G + k
                c_r = csels_ref[r, 0]
                off = pl.multiple_of(c_r * 128, 128)
                offs.append(off)
                chunks.append(s_ref[FOLD * r:FOLD * (r + 1),
                                    pl.ds(off, 128)])
            news, nms, amcs = [], [], []
            for k in range(G):
                r = gi * G + k
                blkv = chunks[k]
                eq = blkv == m[r:r + 1, :]
                am = jnp.min(jnp.min(jnp.where(eq, iota8, BIGCOL), axis=1,
                                     keepdims=True), axis=0, keepdims=True)
                newblk = jnp.where(iota8 == am, NEG, blkv)
                news.append(newblk)
                nms.append(jnp.max(jnp.max(newblk, axis=1, keepdims=True),
                                   axis=0, keepdims=True))
                amcs.append(am)
            for k in range(G):
                r = gi * G + k
                s_ref[FOLD * r:FOLD * (r + 1), pl.ds(offs[k], 128)] = news[k]
                nm_ref[r:r + 1, :] = nms[k]
                amc_ref[r:r + 1, :] = offs[k] * FOLD + amcs[k]

        cm_ref[...] = jnp.where(li == csel, nm_ref[...], cm_ref[...])
        vals_ref[...] = jnp.where(kiota == j, m, vals_ref[...])
        cols_ref[...] = jnp.where(kiota == j, amc_ref[...], cols_ref[...])
        return carry

    lax.fori_loop(0, K_KEEP, extract, 0, unroll=False)

    vals_ref[...] = jnp.where(kiota >= K_KEEP, vals_ref[:, 0:1], vals_ref[...])
    cols_ref[...] = jnp.where(kiota >= K_KEEP, cols_ref[:, 0:1], cols_ref[...])


def _topk_call(x, xnn, interpret=False):
    return pl.pallas_call(
        _topk_body,
        grid=(Q_TOTAL // QB,),
        in_specs=[
            pl.BlockSpec((QB, DIM), lambda i: (i, 0)),
            pl.BlockSpec((DIM, CPAD), lambda i: (0, 0)),
        ],
        out_specs=[
            pl.BlockSpec((QB, K_PAD), lambda i: (i, 0)),
            pl.BlockSpec((QB, K_PAD), lambda i: (i, 0)),
        ],
        out_shape=[
            jax.ShapeDtypeStruct((Q_TOTAL, K_PAD), jnp.float32),
            jax.ShapeDtypeStruct((Q_TOTAL, K_PAD), jnp.int32),
        ],
        scratch_shapes=[
            pltpu.VMEM((S3R, SW), jnp.float32),
            pltpu.VMEM((QB, NCHPAD), jnp.float32),
            pltpu.VMEM((QB, 1), jnp.int32),
            pltpu.VMEM((QB, 1), jnp.float32),
            pltpu.VMEM((QB, 1), jnp.int32),
            pltpu.SMEM((QB, 1), jnp.int32),
            pltpu.SemaphoreType.DMA,
        ],
        interpret=interpret,
    )(x, xnn)


def _prep_call(x_n, interpret=False):
    xp = jnp.pad(x_n, ((0, CPAD - N_KEYS), (0, 0)))
    return pl.pallas_call(
        _prep_body,
        grid=(CPAD // PREP_B,),
        in_specs=[pl.BlockSpec((PREP_B, DIM), lambda i: (i, 0))],
        out_specs=pl.BlockSpec((DIM, PREP_B), lambda i: (0, i)),
        out_shape=jax.ShapeDtypeStruct((DIM, CPAD), jnp.float32),
        interpret=interpret,
    )(xp)


def _make_scatter_kernel():
    mesh = plsc.VectorSubcoreMesh(core_axis_name="c", subcore_axis_name="s")

    @functools.partial(
        pl.kernel,
        out_type=jax.ShapeDtypeStruct((Q_TOTAL, N_KEYS), jnp.float32),
        mesh=mesh,
        compiler_params=pltpu.CompilerParams(needs_layout_passes=False),
        scratch_types=[
            pltpu.VMEM((N_KEYS,), jnp.float32),
            pltpu.VMEM((PERT,), jnp.int32),
            pltpu.VMEM((PERT,), jnp.float32),
        ],
    )
    def scatter_kernel(vals_hbm, idx_hbm, out_hbm, row_v, idx_v, val_v):
        wid = lax.axis_index("s") * 2 + lax.axis_index("c")

        def fill_neg(i, carry):
            row_v[pl.ds(i * 16, 16)] = jnp.full((16,), NEG, jnp.float32)
            return carry

        lax.fori_loop(0, N_KEYS // 16, fill_neg, 0, unroll=False)

        pltpu.sync_copy(idx_hbm.at[wid], idx_v)
        pltpu.sync_copy(vals_hbm.at[wid], val_v)
        negv = jnp.full((16,), NEG, jnp.float32)
        for r in range(RPT):
            for k in range(K_PAD // 16):
                o = r * K_PAD + k * 16
                plsc.store_scatter(row_v, [idx_v[pl.ds(o, 16)]],
                                   val_v[pl.ds(o, 16)])
            pltpu.sync_copy(row_v, out_hbm.at[wid * RPT + r])
            for k in range(K_PAD // 16):
                o = r * K_PAD + k * 16
                plsc.store_scatter(row_v, [idx_v[pl.ds(o, 16)]], negv)

    return scatter_kernel


def kernel(x, x_n):
    xnn = _prep_call(x_n)
    vals, cols = _topk_call(x, xnn)
    vals2 = vals.reshape(NW, PERT)
    idx2 = cols.reshape(NW, PERT)
    out = _make_scatter_kernel()(vals2, idx2)
    return out


# speedup vs baseline: 25.5647x; 1.0673x over previous
import functools

import jax
import jax.numpy as jnp
from jax import lax
from jax.experimental import pallas as pl
from jax.experimental.pallas import tpu as pltpu
from jax.experimental.pallas import tpu_sc as plsc

Q_TOTAL = 1024
N_KEYS = 100000
DIM = 64
K_KEEP = 33
K_PAD = 48

QB = 64
CPAD = 100352
CW = 1024
NCH = CPAD // CW
NCHPAD = 128
MW = 2048
NMM = CPAD // MW
FOLD = 8
SW = CPAD // FOLD
S3R = QB * FOLD
PREP_B = 2048
NEG = float("-inf")
BIGCOL = 2**30

NW = 32
RPT = Q_TOTAL // NW
PERT = RPT * K_PAD


def _prep_body(xn_ref, out_ref):
    v = xn_ref[...]
    nrm = jnp.sqrt(jnp.sum(v * v, axis=1, keepdims=True))
    out_ref[...] = (v / jnp.maximum(nrm, 1e-12)).T


def _topk_body(q_ref, xnn_ref, vals_ref, cols_ref, s_ref, cm_ref, amc_ref,
               nm_ref, cselv_ref, csels_ref, sem):
    q = q_ref[...]
    qn = q / jnp.maximum(jnp.sqrt(jnp.sum(q * q, axis=1, keepdims=True)), 1e-12)

    cm_ref[...] = jnp.full((QB, NCHPAD), NEG, jnp.float32)
    vals_ref[...] = jnp.full((QB, K_PAD), NEG, jnp.float32)
    cols_ref[...] = jnp.zeros((QB, K_PAD), jnp.int32)

    mcol = jax.lax.broadcasted_iota(jnp.int32, (QB, MW), 1)
    li = jax.lax.broadcasted_iota(jnp.int32, (QB, NCHPAD), 1)
    kiota = jax.lax.broadcasted_iota(jnp.int32, (QB, K_PAD), 1)
    iota8 = (jax.lax.broadcasted_iota(jnp.int32, (FOLD, 128), 0) * 128
             + jax.lax.broadcasted_iota(jnp.int32, (FOLD, 128), 1))
    FCH = MW // (FOLD * 128)

    def mm(c, carry):
        off = pl.multiple_of(c * MW, MW)
        blk = xnn_ref[:, pl.ds(off, MW)]
        sim = jax.lax.dot_general(
            qn, blk, (((1,), (0,)), ((), ())),
            preferred_element_type=jnp.float32)
        sim = jnp.where(c * MW + mcol < N_KEYS, sim, NEG)
        folded = sim.reshape(QB, FCH, FOLD, 128).transpose(0, 2, 1, 3)
        s_ref[:, pl.ds(pl.multiple_of(c * (MW // FOLD), MW // FOLD),
                       MW // FOLD)] = folded.reshape(S3R, MW // FOLD)
        cmu = cm_ref[...]
        for sub in range(MW // CW):
            mx = jnp.max(sim[:, sub * CW:(sub + 1) * CW], axis=1,
                         keepdims=True)
            cmu = jnp.where(li == c * (MW // CW) + sub, mx, cmu)
        cm_ref[...] = cmu
        return carry

    lax.fori_loop(0, NMM, mm, 0, unroll=False)

    def extract(j, carry):
        cmv = cm_ref[...]
        m = jnp.max(cmv, axis=1, keepdims=True)
        csel = jnp.min(jnp.where(cmv == m, li, BIGCOL), axis=1, keepdims=True)
        cselv_ref[...] = csel
        pltpu.make_async_copy(cselv_ref, csels_ref, sem).start()
        vals_ref[...] = jnp.where(kiota == j, m, vals_ref[...])
        cols_ref[...] = jnp.where(
            jnp.logical_and(kiota == j - 1, j > 0), amc_ref[...],
            cols_ref[...])
        pltpu.make_async_copy(cselv_ref, csels_ref, sem).wait()

        G = 32
        for gi in range(QB // G):
            offs, chunks = [], []
            for k in range(G):
                r = gi * G + k
                c_r = csels_ref[r, 0]
                off = pl.multiple_of(c_r * 128, 128)
                offs.append(off)
                chunks.append(s_ref[FOLD * r:FOLD * (r + 1),
                                    pl.ds(off, 128)])
            news, nms, amcs = [], [], []
            for k in range(G):
                r = gi * G + k
                blkv = chunks[k]
                eq = blkv == m[r:r + 1, :]
                am = jnp.min(jnp.min(jnp.where(eq, iota8, BIGCOL), axis=1,
                                     keepdims=True), axis=0, keepdims=True)
                newblk = jnp.where(iota8 == am, NEG, blkv)
                news.append(newblk)
                nms.append(jnp.max(jnp.max(newblk, axis=1, keepdims=True),
                                   axis=0, keepdims=True))
                amcs.append(am)
            for k in range(G):
                r = gi * G + k
                s_ref[FOLD * r:FOLD * (r + 1), pl.ds(offs[k], 128)] = news[k]
                nm_ref[r:r + 1, :] = nms[k]
                amc_ref[r:r + 1, :] = offs[k] * FOLD + amcs[k]

        cm_ref[...] = jnp.where(li == csel, nm_ref[...], cm_ref[...])
        return carry

    lax.fori_loop(0, K_KEEP, extract, 0, unroll=False)
    cols_ref[...] = jnp.where(kiota == K_KEEP - 1, amc_ref[...], cols_ref[...])

    vals_ref[...] = jnp.where(kiota >= K_KEEP, vals_ref[:, 0:1], vals_ref[...])
    cols_ref[...] = jnp.where(kiota >= K_KEEP, cols_ref[:, 0:1], cols_ref[...])


def _topk_call(x, xnn, interpret=False):
    return pl.pallas_call(
        _topk_body,
        grid=(Q_TOTAL // QB,),
        in_specs=[
            pl.BlockSpec((QB, DIM), lambda i: (i, 0)),
            pl.BlockSpec((DIM, CPAD), lambda i: (0, 0)),
        ],
        out_specs=[
            pl.BlockSpec((QB, K_PAD), lambda i: (i, 0)),
            pl.BlockSpec((QB, K_PAD), lambda i: (i, 0)),
        ],
        out_shape=[
            jax.ShapeDtypeStruct((Q_TOTAL, K_PAD), jnp.float32),
            jax.ShapeDtypeStruct((Q_TOTAL, K_PAD), jnp.int32),
        ],
        scratch_shapes=[
            pltpu.VMEM((S3R, SW), jnp.float32),
            pltpu.VMEM((QB, NCHPAD), jnp.float32),
            pltpu.VMEM((QB, 1), jnp.int32),
            pltpu.VMEM((QB, 1), jnp.float32),
            pltpu.VMEM((QB, 1), jnp.int32),
            pltpu.SMEM((QB, 1), jnp.int32),
            pltpu.SemaphoreType.DMA,
        ],
        interpret=interpret,
    )(x, xnn)


def _prep_call(x_n, interpret=False):
    xp = jnp.pad(x_n, ((0, CPAD - N_KEYS), (0, 0)))
    return pl.pallas_call(
        _prep_body,
        grid=(CPAD // PREP_B,),
        in_specs=[pl.BlockSpec((PREP_B, DIM), lambda i: (i, 0))],
        out_specs=pl.BlockSpec((DIM, PREP_B), lambda i: (0, i)),
        out_shape=jax.ShapeDtypeStruct((DIM, CPAD), jnp.float32),
        interpret=interpret,
    )(xp)


def _make_scatter_kernel():
    mesh = plsc.VectorSubcoreMesh(core_axis_name="c", subcore_axis_name="s")

    @functools.partial(
        pl.kernel,
        out_type=jax.ShapeDtypeStruct((Q_TOTAL, N_KEYS), jnp.float32),
        mesh=mesh,
        compiler_params=pltpu.CompilerParams(needs_layout_passes=False),
        scratch_types=[
            pltpu.VMEM((N_KEYS,), jnp.float32),
            pltpu.VMEM((PERT,), jnp.int32),
            pltpu.VMEM((PERT,), jnp.float32),
        ],
    )
    def scatter_kernel(vals_hbm, idx_hbm, out_hbm, row_v, idx_v, val_v):
        wid = lax.axis_index("s") * 2 + lax.axis_index("c")

        def fill_neg(i, carry):
            row_v[pl.ds(i * 16, 16)] = jnp.full((16,), NEG, jnp.float32)
            return carry

        lax.fori_loop(0, N_KEYS // 16, fill_neg, 0, unroll=False)

        pltpu.sync_copy(idx_hbm.at[wid], idx_v)
        pltpu.sync_copy(vals_hbm.at[wid], val_v)
        negv = jnp.full((16,), NEG, jnp.float32)
        for r in range(RPT):
            for k in range(K_PAD // 16):
                o = r * K_PAD + k * 16
                plsc.store_scatter(row_v, [idx_v[pl.ds(o, 16)]],
                                   val_v[pl.ds(o, 16)])
            pltpu.sync_copy(row_v, out_hbm.at[wid * RPT + r])
            for k in range(K_PAD // 16):
                o = r * K_PAD + k * 16
                plsc.store_scatter(row_v, [idx_v[pl.ds(o, 16)]], negv)

    return scatter_kernel


def kernel(x, x_n):
    xnn = _prep_call(x_n)
    vals, cols = _topk_call(x, xnn)
    vals2 = vals.reshape(NW, PERT)
    idx2 = cols.reshape(NW, PERT)
    out = _make_scatter_kernel()(vals2, idx2)
    return out


# speedup vs baseline: 26.1744x; 1.0239x over previous
import functools

import jax
import jax.numpy as jnp
from jax import lax
from jax.experimental import pallas as pl
from jax.experimental.pallas import tpu as pltpu
from jax.experimental.pallas import tpu_sc as plsc

Q_TOTAL = 1024
N_KEYS = 100000
DIM = 64
K_KEEP = 33
K_PAD = 48

QB = 64
CPAD = 100352
CW = 1024
NCH = CPAD // CW
NCHPAD = 128
MW = 2048
NMM = CPAD // MW
FOLD = 8
SW = CPAD // FOLD
S3R = QB * FOLD
PREP_B = 2048
NEG = float("-inf")
BIGCOL = 2**30

NW = 32
RPT = Q_TOTAL // NW
PERT = RPT * K_PAD


def _prep_body(xn_ref, out_ref):
    v = xn_ref[...]
    nrm = jnp.sqrt(jnp.sum(v * v, axis=1, keepdims=True))
    out_ref[...] = (v / jnp.maximum(nrm, 1e-12)).T


def _topk_body(q_ref, xnn_ref, vals_ref, cols_ref, s_ref, cm_ref, amc_ref,
               nm_ref, cselv_ref, csels_ref, sems):
    q = q_ref[...]
    qn = q / jnp.maximum(jnp.sqrt(jnp.sum(q * q, axis=1, keepdims=True)), 1e-12)

    cm_ref[...] = jnp.full((QB, NCHPAD), NEG, jnp.float32)
    vals_ref[...] = jnp.full((QB, K_PAD), NEG, jnp.float32)
    cols_ref[...] = jnp.zeros((QB, K_PAD), jnp.int32)

    mcol = jax.lax.broadcasted_iota(jnp.int32, (QB, MW), 1)
    li = jax.lax.broadcasted_iota(jnp.int32, (QB, NCHPAD), 1)
    kiota = jax.lax.broadcasted_iota(jnp.int32, (QB, K_PAD), 1)
    iota8 = (jax.lax.broadcasted_iota(jnp.int32, (FOLD, 128), 0) * 128
             + jax.lax.broadcasted_iota(jnp.int32, (FOLD, 128), 1))
    FCH = MW // (FOLD * 128)

    def mm(c, carry):
        off = pl.multiple_of(c * MW, MW)
        blk = xnn_ref[:, pl.ds(off, MW)]
        sim = jax.lax.dot_general(
            qn, blk, (((1,), (0,)), ((), ())),
            preferred_element_type=jnp.float32)
        sim = jnp.where(c * MW + mcol < N_KEYS, sim, NEG)
        folded = sim.reshape(QB, FCH, FOLD, 128).transpose(0, 2, 1, 3)
        s_ref[:, pl.ds(pl.multiple_of(c * (MW // FOLD), MW // FOLD),
                       MW // FOLD)] = folded.reshape(S3R, MW // FOLD)
        cmu = cm_ref[...]
        for sub in range(MW // CW):
            mx = jnp.max(sim[:, sub * CW:(sub + 1) * CW], axis=1,
                         keepdims=True)
            cmu = jnp.where(li == c * (MW // CW) + sub, mx, cmu)
        cm_ref[...] = cmu
        return carry

    lax.fori_loop(0, NMM, mm, 0, unroll=False)

    HB = QB // 2
    liH = jax.lax.broadcasted_iota(jnp.int32, (HB, NCHPAD), 1)

    def half_prefix(h):
        lo = h * HB
        cmv = cm_ref[lo:lo + HB, :]
        m = jnp.max(cmv, axis=1, keepdims=True)
        csel = jnp.min(jnp.where(cmv == m, liH, BIGCOL), axis=1,
                       keepdims=True)
        cselv_ref[lo:lo + HB, :] = csel
        cp = pltpu.make_async_copy(cselv_ref.at[pl.ds(lo, HB)],
                                   csels_ref.at[pl.ds(lo, HB)], sems.at[h])
        cp.start()
        return m, csel, cp

    def half_rows(h, m):
        lo = h * HB
        offs, chunks = [], []
        for k in range(HB):
            r = lo + k
            c_r = csels_ref[r, 0]
            off = pl.multiple_of(c_r * 128, 128)
            offs.append(off)
            chunks.append(s_ref[FOLD * r:FOLD * (r + 1), pl.ds(off, 128)])
        news, nms, amcs = [], [], []
        for k in range(HB):
            blkv = chunks[k]
            eq = blkv == m[k:k + 1, :]
            am = jnp.min(jnp.min(jnp.where(eq, iota8, BIGCOL), axis=1,
                                 keepdims=True), axis=0, keepdims=True)
            newblk = jnp.where(iota8 == am, NEG, blkv)
            news.append(newblk)
            nms.append(jnp.max(jnp.max(newblk, axis=1, keepdims=True),
                               axis=0, keepdims=True))
            amcs.append(am)
        for k in range(HB):
            r = lo + k
            s_ref[FOLD * r:FOLD * (r + 1), pl.ds(offs[k], 128)] = news[k]
            nm_ref[r:r + 1, :] = nms[k]
            amc_ref[r:r + 1, :] = offs[k] * FOLD + amcs[k]

    def extract(j, carry):
        mA, cselA, cpA = half_prefix(0)
        mB, cselB, cpB = half_prefix(1)
        m = jnp.concatenate([mA, mB], axis=0)
        vals_ref[...] = jnp.where(kiota == j, m, vals_ref[...])
        cols_ref[...] = jnp.where(
            jnp.logical_and(kiota == j - 1, j > 0), amc_ref[...],
            cols_ref[...])
        cpA.wait()
        half_rows(0, mA)
        cm_ref[0:HB, :] = jnp.where(liH == cselA, nm_ref[0:HB, :],
                                    cm_ref[0:HB, :])
        cpB.wait()
        half_rows(1, mB)
        cm_ref[HB:QB, :] = jnp.where(liH == cselB, nm_ref[HB:QB, :],
                                     cm_ref[HB:QB, :])
        return carry

    lax.fori_loop(0, K_KEEP, extract, 0, unroll=False)
    cols_ref[...] = jnp.where(kiota == K_KEEP - 1, amc_ref[...], cols_ref[...])

    vals_ref[...] = jnp.where(kiota >= K_KEEP, vals_ref[:, 0:1], vals_ref[...])
    cols_ref[...] = jnp.where(kiota >= K_KEEP, cols_ref[:, 0:1], cols_ref[...])


def _topk_call(x, xnn, interpret=False):
    return pl.pallas_call(
        _topk_body,
        grid=(Q_TOTAL // QB,),
        in_specs=[
            pl.BlockSpec((QB, DIM), lambda i: (i, 0)),
            pl.BlockSpec((DIM, CPAD), lambda i: (0, 0)),
        ],
        out_specs=[
            pl.BlockSpec((QB, K_PAD), lambda i: (i, 0)),
            pl.BlockSpec((QB, K_PAD), lambda i: (i, 0)),
        ],
        out_shape=[
            jax.ShapeDtypeStruct((Q_TOTAL, K_PAD), jnp.float32),
            jax.ShapeDtypeStruct((Q_TOTAL, K_PAD), jnp.int32),
        ],
        scratch_shapes=[
            pltpu.VMEM((S3R, SW), jnp.float32),
            pltpu.VMEM((QB, NCHPAD), jnp.float32),
            pltpu.VMEM((QB, 1), jnp.int32),
            pltpu.VMEM((QB, 1), jnp.float32),
            pltpu.VMEM((QB, 1), jnp.int32),
            pltpu.SMEM((QB, 1), jnp.int32),
            pltpu.SemaphoreType.DMA((2,)),
        ],
        interpret=interpret,
    )(x, xnn)


def _prep_call(x_n, interpret=False):
    xp = jnp.pad(x_n, ((0, CPAD - N_KEYS), (0, 0)))
    return pl.pallas_call(
        _prep_body,
        grid=(CPAD // PREP_B,),
        in_specs=[pl.BlockSpec((PREP_B, DIM), lambda i: (i, 0))],
        out_specs=pl.BlockSpec((DIM, PREP_B), lambda i: (0, i)),
        out_shape=jax.ShapeDtypeStruct((DIM, CPAD), jnp.float32),
        interpret=interpret,
    )(xp)


def _make_scatter_kernel():
    mesh = plsc.VectorSubcoreMesh(core_axis_name="c", subcore_axis_name="s")

    @functools.partial(
        pl.kernel,
        out_type=jax.ShapeDtypeStruct((Q_TOTAL, N_KEYS), jnp.float32),
        mesh=mesh,
        compiler_params=pltpu.CompilerParams(needs_layout_passes=False),
        scratch_types=[
            pltpu.VMEM((N_KEYS,), jnp.float32),
            pltpu.VMEM((PERT,), jnp.int32),
            pltpu.VMEM((PERT,), jnp.float32),
        ],
    )
    def scatter_kernel(vals_hbm, idx_hbm, out_hbm, row_v, idx_v, val_v):
        wid = lax.axis_index("s") * 2 + lax.axis_index("c")

        def fill_neg(i, carry):
            row_v[pl.ds(i * 16, 16)] = jnp.full((16,), NEG, jnp.float32)
            return carry

        lax.fori_loop(0, N_KEYS // 16, fill_neg, 0, unroll=False)

        pltpu.sync_copy(idx_hbm.at[wid], idx_v)
        pltpu.sync_copy(vals_hbm.at[wid], val_v)
        negv = jnp.full((16,), NEG, jnp.float32)
        for r in range(RPT):
            for k in range(K_PAD // 16):
                o = r * K_PAD + k * 16
                plsc.store_scatter(row_v, [idx_v[pl.ds(o, 16)]],
                                   val_v[pl.ds(o, 16)])
            pltpu.sync_copy(row_v, out_hbm.at[wid * RPT + r])
            for k in range(K_PAD // 16):
                o = r * K_PAD + k * 16
                plsc.store_scatter(row_v, [idx_v[pl.ds(o, 16)]], negv)

    return scatter_kernel


def kernel(x, x_n):
    xnn = _prep_call(x_n)
    vals, cols = _topk_call(x, xnn)
    vals2 = vals.reshape(NW, PERT)
    idx2 = cols.reshape(NW, PERT)
    out = _make_scatter_kernel()(vals2, idx2)
    return out
